# Initial kernel scaffold; baseline (speedup 1.0000x reference)
import jax
import jax.numpy as jnp
from jax.experimental import pallas as pl


def kernel(pred_class, pred_bbox_deltas, anchors):
    raise NotImplementedError("write your pallas kernel here")


# speedup vs baseline: 91.2388x; 91.2388x over previous
import jax
import jax.numpy as jnp
from jax.experimental import pallas as pl

B = 4
N = 20000
PRE_NMS = 2000
POST_NMS = 100
NMS_THRESH = 0.7
MIN_SIZE = 0.001
IMG_W = 1024.0
IMG_H = 1024.0
BBOX_XFORM_CLIP = 4.135166556742356

C = 128
R = 160
PAD_N = R * C
NEG = -1e9


def _nms_kernel(sc_ref, dx_ref, dy_ref, dw_ref, dh_ref,
                ax1_ref, ay1_ref, ax2_ref, ay2_ref,
                ox1_ref, oy1_ref, ox2_ref, oy2_ref, osc_ref):
    s_raw = sc_ref[...]
    fr = jax.lax.broadcasted_iota(jnp.int32, (R, C), 0)
    fc = jax.lax.broadcasted_iota(jnp.int32, (R, C), 1)
    flat = fr * C + fc
    flat3 = flat[None]
    inb = (flat3 < N)

    u = jax.lax.bitcast_convert_type(s_raw, jnp.uint32)
    neg = u >= jnp.uint32(0x80000000)
    sortk = u ^ jnp.where(neg, jnp.uint32(0xFFFFFFFF), jnp.uint32(0x80000000))
    sortk = jnp.where(inb, sortk, jnp.uint32(0))

    thr = jnp.zeros((B, 1, 1), jnp.uint32)
    for b in range(32):
        cand = thr | jnp.uint32(1 << (31 - b))
        cnt = jnp.sum((sortk >= cand).astype(jnp.int32), axis=(1, 2),
                      keepdims=True)
        thr = jnp.where(cnt >= PRE_NMS, cand, thr)

    gt = sortk > thr
    cnt_gt = jnp.sum(gt.astype(jnp.float32), axis=(1, 2), keepdims=True)
    tie = sortk == thr
    tie_f = tie.astype(jnp.float32)
    iC0 = jax.lax.broadcasted_iota(jnp.int32, (C, C), 0)
    iC1 = jax.lax.broadcasted_iota(jnp.int32, (C, C), 1)
    ltC = (iC0 < iC1).astype(jnp.float32)
    iR0 = jax.lax.broadcasted_iota(jnp.int32, (R, R), 0)
    iR1 = jax.lax.broadcasted_iota(jnp.int32, (R, R), 1)
    ltR = (iR0 < iR1).astype(jnp.float32)
    inrow = jax.lax.dot_general(
        tie_f.reshape(B * R, C), ltC, (((1,), (0,)), ((), ())),
        preferred_element_type=jnp.float32).reshape(B, R, C)
    rowsum = jnp.sum(tie_f, axis=2)
    rowpre = jax.lax.dot_general(
        rowsum, ltR, (((1,), (0,)), ((), ())),
        preferred_element_type=jnp.float32)
    tie_rank = rowpre[:, :, None] + inrow
    need = PRE_NMS - cnt_gt
    in_topk = gt | (tie & (tie_rank < need))

    ax1 = ax1_ref[...][None]
    ay1 = ay1_ref[...][None]
    ax2 = ax2_ref[...][None]
    ay2 = ay2_ref[...][None]
    w = ax2 - ax1
    h = ay2 - ay1
    cx = ax1 + 0.5 * w
    cy = ay1 + 0.5 * h
    pcx = dx_ref[...] * w + cx
    pcy = dy_ref[...] * h + cy
    pw = jnp.exp(jnp.minimum(dw_ref[...], BBOX_XFORM_CLIP)) * w
    ph = jnp.exp(jnp.minimum(dh_ref[...], BBOX_XFORM_CLIP)) * h
    x1c = jnp.clip(pcx - 0.5 * pw, 0.0, IMG_W)
    y1c = jnp.clip(pcy - 0.5 * ph, 0.0, IMG_H)
    x2c = jnp.clip(pcx + 0.5 * pw, 0.0, IMG_W)
    y2c = jnp.clip(pcy + 0.5 * ph, 0.0, IMG_H)
    validb = ((x2c - x1c) >= MIN_SIZE) & ((y2c - y1c) >= MIN_SIZE)
    area = (x2c - x1c) * (y2c - y1c)

    s0 = jnp.where(in_topk & validb & inb, s_raw, NEG)

    kcol = jax.lax.broadcasted_iota(jnp.int32, (1, POST_NMS), 1)
    accz = jnp.zeros((B, POST_NMS), jnp.float32)

    def nms_body(k, carry):
        s, a1, b1, a2, b2, asc = carry
        m = jnp.max(s, axis=(1, 2), keepdims=True)
        idx = jnp.min(jnp.where(s == m, flat3, PAD_N), axis=(1, 2),
                      keepdims=True)
        onehot = flat3 == idx
        bx1 = jnp.sum(jnp.where(onehot, x1c, 0.0), axis=(1, 2), keepdims=True)
        by1 = jnp.sum(jnp.where(onehot, y1c, 0.0), axis=(1, 2), keepdims=True)
        bx2 = jnp.sum(jnp.where(onehot, x2c, 0.0), axis=(1, 2), keepdims=True)
        by2 = jnp.sum(jnp.where(onehot, y2c, 0.0), axis=(1, 2), keepdims=True)
        good = m > -1e8
        sel = kcol == k
        a1 = a1 + jnp.where(sel & good[:, 0], bx1[:, 0], 0.0)
        b1 = b1 + jnp.where(sel & good[:, 0], by1[:, 0], 0.0)
        a2 = a2 + jnp.where(sel & good[:, 0], bx2[:, 0], 0.0)
        b2 = b2 + jnp.where(sel & good[:, 0], by2[:, 0], 0.0)
        asc = asc + jnp.where(sel & good[:, 0], m[:, 0], 0.0)
        lx = jnp.maximum(x1c, bx1)
        ly = jnp.maximum(y1c, by1)
        rx = jnp.minimum(x2c, bx2)
        ry = jnp.minimum(y2c, by2)
        inter = jnp.maximum(rx - lx, 0.0) * jnp.maximum(ry - ly, 0.0)
        barea = (bx2 - bx1) * (by2 - by1)
        iou = inter / (area + barea - inter + 1e-9)
        s = jnp.where((iou > NMS_THRESH) | onehot, NEG, s)
        return s, a1, b1, a2, b2, asc

    _, a1, b1, a2, b2, asc = jax.lax.fori_loop(
        0, POST_NMS, nms_body, (s0, accz, accz, accz, accz, accz))
    ox1_ref[...] = a1
    oy1_ref[...] = b1
    ox2_ref[...] = a2
    oy2_ref[...] = b2
    osc_ref[...] = asc


def _pad_rc(x2d):
    return jnp.pad(x2d, ((0, 0), (0, PAD_N - N))).reshape(B, R, C)


def kernel(pred_class, pred_bbox_deltas, anchors):
    scores = _pad_rc(pred_class[:, :, 0])
    dx = _pad_rc(pred_bbox_deltas[:, :, 0])
    dy = _pad_rc(pred_bbox_deltas[:, :, 1])
    dw = _pad_rc(pred_bbox_deltas[:, :, 2])
    dh = _pad_rc(pred_bbox_deltas[:, :, 3])
    apad = jnp.pad(anchors, ((0, PAD_N - N), (0, 0)))
    ax1 = apad[:, 0].reshape(R, C)
    ay1 = apad[:, 1].reshape(R, C)
    ax2 = apad[:, 2].reshape(R, C)
    ay2 = apad[:, 3].reshape(R, C)

    f = jax.ShapeDtypeStruct((B, POST_NMS), jnp.float32)
    ox1, oy1, ox2, oy2, osc = pl.pallas_call(
        _nms_kernel,
        out_shape=[f, f, f, f, f],
    )(scores, dx, dy, dw, dh, ax1, ay1, ax2, ay2)

    boxes = jnp.stack([ox1, oy1, ox2, oy2], axis=2)
    return boxes, osc
